# Initial kernel scaffold; baseline (speedup 1.0000x reference)
import jax
import jax.numpy as jnp
from jax.experimental import pallas as pl


def kernel(x, edge_index, Wl_s1, bl_s1, Wr_s1, br_s1, att_s1, b_s1, Wl_s2, bl_s2, Wr_s2, br_s2, att_s2, b_s2, Wl_p1, bl_p1, Wr_p1, br_p1, att_p1, b_p1, Wl_p2, bl_p2, Wr_p2, br_p2, att_p2, b_p2):
    raise NotImplementedError("write your pallas kernel here")


# speedup vs baseline: 2.3223x; 2.3223x over previous
import functools

import jax
import jax.numpy as jnp
from jax import lax
from jax.experimental import pallas as pl
from jax.experimental.pallas import tpu as pltpu
from jax.experimental.pallas import tpu_sc as plsc

N = 10000
E = 160000
IN_DIM = 256
HIDDEN = 128
HEADS = 4
S_DIM = 64
P_DIM = 64
ETA = 1e-6
NEG_SLOPE = 0.2

E2 = E + N
BE = 128
E2P = E2 + BE
R = 320
NW = 32
D = 128
NC = 2
BR = 1000


def _m1_body(x_ref, w_ref, b_ref, o_ref):
    acc = jnp.dot(x_ref[...], w_ref[...], preferred_element_type=jnp.float32)
    o_ref[0] = acc + b_ref[0, 0:1, :]


def _proj1(x, wcat, bcat, nt):
    k = x.shape[1]
    return pl.pallas_call(
        _m1_body,
        grid=(nt, N // BR),
        in_specs=[
            pl.BlockSpec((BR, k), lambda i, j: (j, 0)),
            pl.BlockSpec((k, D), lambda i, j: (0, i)),
            pl.BlockSpec((1, 8, D), lambda i, j: (i, 0, 0)),
        ],
        out_specs=pl.BlockSpec((1, BR, D), lambda i, j: (i, j, 0)),
        out_shape=jax.ShapeDtypeStruct((nt, N, D), jnp.float32),
    )(x, wcat, bcat)


def _m2_body(s_ref, w_ref, b_ref, o_ref):
    acc = jnp.dot(
        s_ref[0], w_ref[pl.ds(0, D), :], preferred_element_type=jnp.float32
    )
    for h in range(1, HEADS):
        acc += jnp.dot(
            s_ref[h], w_ref[pl.ds(h * D, D), :],
            preferred_element_type=jnp.float32,
        )
    o_ref[0] = acc + b_ref[0, 0:1, :]


def _proj2(s4, wcat, bcat):
    return pl.pallas_call(
        _m2_body,
        grid=(2, N // BR),
        in_specs=[
            pl.BlockSpec((HEADS, BR, D), lambda i, j: (0, j, 0)),
            pl.BlockSpec((HEADS * D, D), lambda i, j: (0, i)),
            pl.BlockSpec((1, 8, D), lambda i, j: (i, 0, 0)),
        ],
        out_specs=pl.BlockSpec((1, BR, D), lambda i, j: (i, j, 0)),
        out_shape=jax.ShapeDtypeStruct((2, N, D), jnp.float32),
    )(s4, wcat, bcat)


def _fin_body(s_ref, p_ref, o1_ref, o2_ref):
    o1_ref[:, :S_DIM] = s_ref[:, :S_DIM]
    o1_ref[:, S_DIM:] = p_ref[:, :P_DIM]
    o2_ref[:, :S_DIM] = jax.nn.softplus(s_ref[:, S_DIM:]) + ETA
    o2_ref[:, S_DIM:] = jax.nn.softplus(p_ref[:, P_DIM:]) + ETA


def _finalize(s, p):
    return pl.pallas_call(
        _fin_body,
        grid=(N // BR,),
        in_specs=[
            pl.BlockSpec((BR, D), lambda i: (i, 0)),
            pl.BlockSpec((BR, D), lambda i: (i, 0)),
        ],
        out_specs=[
            pl.BlockSpec((BR, D), lambda i: (i, 0)),
            pl.BlockSpec((BR, D), lambda i: (i, 0)),
        ],
        out_shape=[
            jax.ShapeDtypeStruct((N, D), jnp.float32),
            jax.ShapeDtypeStruct((N, D), jnp.float32),
        ],
    )(s, p)


def _edge_body(hp_n, tbl_ref, src_ref, dst_ref, eb_ref, tix_ref,
               att_ref, bias_ref, flag_ref, out_ref,
               ebv, tixv, attv, biasv, flagv,
               xrv, accv, denv, srcb, dstb, gidx, msgv, sem):
    wid = lax.axis_index("s") * NC + lax.axis_index("c")
    base_own = wid * R
    base_x = jnp.minimum(base_own, N - R)
    pltpu.sync_copy(eb_ref.at[wid], ebv)
    ebvec = ebv[...]
    e0 = ebvec[0] & ~jnp.int32(BE - 1)
    e1 = ebvec[1]
    nb = (e1 - e0 + (BE - 1)) // BE

    def hp_loop(hp, _):
        pltpu.sync_copy(tix_ref.at[hp], tixv)
        tvec = tixv[...]
        xli = tvec[0]
        xri = tvec[1]
        pltpu.sync_copy(att_ref.at[hp], attv)
        pltpu.sync_copy(bias_ref.at[hp], biasv)
        pltpu.sync_copy(flag_ref.at[hp], flagv)
        pltpu.sync_copy(tbl_ref.at[pl.ds(pl.multiple_of(xri * N + base_x, 8), R), :], xrv)

        zero16 = jnp.zeros((16,), jnp.float32)

        def zero_loop(r, _):
            for c in range(8):
                accv[r, pl.ds(c * 16, 16)] = zero16
            return 0

        lax.fori_loop(0, R, zero_loop, 0)
        for i in range(20):
            denv[pl.ds(i * 16, 16)] = zero16

        tbase = xli * N

        def blk_loop(b, _):
            off = pl.multiple_of(e0 + b * BE, BE)
            pltpu.sync_copy(src_ref.at[pl.ds(off, BE)], srcb)
            pltpu.sync_copy(dst_ref.at[pl.ds(off, BE)], dstb)
            for c in range(8):
                gidx[pl.ds(c * 16, 16)] = srcb[pl.ds(c * 16, 16)] + tbase
            pltpu.async_copy(tbl_ref.at[gidx], msgv, sem).wait()
            for c in range(8):
                erow = lax.iota(jnp.int32, 16) + (c * 16)
                d16 = dstb[pl.ds(c * 16, 16)]
                mask = (d16 >= base_own) & (d16 < base_own + R)
                dl = jnp.clip(d16 - base_x, 0, R - 1)

                def alpha_loop(k, alpha):
                    colk = jnp.full((16,), k, jnp.int32)
                    mv = plsc.load_gather(msgv, [erow, colk])
                    xv = plsc.load_gather(xrv, [dl, colk])
                    v = mv + xv
                    v = jnp.where(v > 0, v, NEG_SLOPE * v)
                    av = plsc.load_gather(attv, [colk])
                    return alpha + v * av

                alpha = lax.fori_loop(0, D, alpha_loop, zero16)
                w = jnp.where(mask, jnp.exp(alpha), 0.0)
                plsc.addupdate_scatter(denv, [dl], w, mask=mask)

                def acc_loop(k, _):
                    colk = jnp.full((16,), k, jnp.int32)
                    mv = plsc.load_gather(msgv, [erow, colk])
                    plsc.addupdate_scatter(accv, [dl, colk], w * mv, mask=mask)
                    return 0

                lax.fori_loop(0, D, acc_loop, 0)
            return 0

        lax.fori_loop(0, nb, blk_loop, 0)

        def fin_loop(r, _):
            r16 = jnp.full((16,), r, jnp.int32)
            rin = 1.0 / plsc.load_gather(denv, [r16])
            for c in range(8):
                v = accv[r, pl.ds(c * 16, 16)]
                bv = biasv[pl.ds(c * 16, 16)]
                fv = flagv[pl.ds(c * 16, 16)]
                v = v * rin + bv
                accv[r, pl.ds(c * 16, 16)] = jnp.maximum(v, fv * v)
            return 0

        lax.fori_loop(0, R, fin_loop, 0)

        @pl.when(wid < NW - 1)
        def _():
            pltpu.sync_copy(accv, out_ref.at[hp, pl.ds(pl.multiple_of(base_own, 8), R), :])

        @pl.when(wid == NW - 1)
        def _():
            nlast = N - (NW - 1) * R
            pltpu.sync_copy(
                accv.at[pl.ds(R - nlast, nlast), :],
                out_ref.at[hp, pl.ds((NW - 1) * R, nlast), :],
            )

        return 0

    lax.fori_loop(0, hp_n, hp_loop, 0)


def _edge_phase(hp_n, tbl, src_s, dst_s, eb, tix, att, bias, flag):
    mesh = plsc.VectorSubcoreMesh(core_axis_name="c", subcore_axis_name="s")
    fn = pl.kernel(
        functools.partial(_edge_body, hp_n),
        mesh=mesh,
        compiler_params=pltpu.CompilerParams(needs_layout_passes=False),
        out_type=jax.ShapeDtypeStruct((hp_n, N, D), jnp.float32),
        scratch_types=[
            pltpu.VMEM((16,), jnp.int32),
            pltpu.VMEM((16,), jnp.int32),
            pltpu.VMEM((D,), jnp.float32),
            pltpu.VMEM((D,), jnp.float32),
            pltpu.VMEM((D,), jnp.float32),
            pltpu.VMEM((R, D), jnp.float32),
            pltpu.VMEM((R, D), jnp.float32),
            pltpu.VMEM((320,), jnp.float32),
            pltpu.VMEM((BE,), jnp.int32),
            pltpu.VMEM((BE,), jnp.int32),
            pltpu.VMEM((BE,), jnp.int32),
            pltpu.VMEM((BE, D), jnp.float32),
            pltpu.SemaphoreType.DMA,
        ],
    )
    return fn(tbl, src_s, dst_s, eb, tix, att, bias, flag)


def kernel(x, edge_index, Wl_s1, bl_s1, Wr_s1, br_s1, att_s1, b_s1, Wl_s2, bl_s2, Wr_s2, br_s2, att_s2, b_s2, Wl_p1, bl_p1, Wr_p1, br_p1, att_p1, b_p1, Wl_p2, bl_p2, Wr_p2, br_p2, att_p2, b_p2):
    loop = jnp.arange(N, dtype=edge_index.dtype)
    src = jnp.concatenate([edge_index[0], loop])
    dst = jnp.concatenate([edge_index[1], loop])
    perm = jnp.argsort(dst)
    dst_s = dst[perm]
    src_s = src[perm]
    src_s = jnp.concatenate([src_s, jnp.zeros((E2P - E2,), jnp.int32)])
    dst_s = jnp.concatenate(
        [dst_s, jnp.full((E2P - E2,), jnp.int32(1 << 20))]
    )
    bounds = jnp.arange(0, NW + 1, dtype=jnp.int32) * R
    eb = jnp.searchsorted(dst_s[:E2], bounds).astype(jnp.int32)
    ebm = jnp.stack([eb[:NW], eb[1:]], axis=1)
    ebm = jnp.pad(ebm, ((0, 0), (0, 14)))

    w1 = jnp.concatenate([Wl_s1, Wr_s1, Wl_p1, Wr_p1], axis=1)
    b1 = jnp.concatenate([bl_s1, br_s1, bl_p1, br_p1]).reshape(16, 1, D)
    b1 = jnp.broadcast_to(b1, (16, 8, D))
    h1 = _proj1(x, w1, b1, 16)

    tixA = jnp.pad(
        jnp.array(
            [[0, 4], [1, 5], [2, 6], [3, 7],
             [8, 12], [9, 13], [10, 14], [11, 15]], jnp.int32
        ),
        ((0, 0), (0, 14)),
    )
    attA = jnp.concatenate([att_s1, att_p1], axis=0)
    biasA = jnp.concatenate([b_s1.reshape(4, D), b_p1.reshape(4, D)], axis=0)
    flagA = jnp.concatenate(
        [jnp.zeros((4, D), jnp.float32), jnp.ones((4, D), jnp.float32)]
    )
    outA = _edge_phase(
        8, h1.reshape(16 * N, D), src_s, dst_s, ebm, tixA, attA, biasA, flagA
    )

    w2s = jnp.concatenate([Wl_s2, Wr_s2], axis=1)
    b2s = jnp.concatenate([bl_s2, br_s2]).reshape(2, 1, D)
    b2s = jnp.broadcast_to(b2s, (2, 8, D))
    w2p = jnp.concatenate([Wl_p2, Wr_p2], axis=1)
    b2p = jnp.concatenate([bl_p2, br_p2]).reshape(2, 1, D)
    b2p = jnp.broadcast_to(b2p, (2, 8, D))
    s2 = _proj2(outA[0:4], w2s, b2s)
    p2 = _proj2(outA[4:8], w2p, b2p)
    tblB = jnp.concatenate([s2, p2], axis=0).reshape(4 * N, D)

    tixB = jnp.pad(jnp.array([[0, 1], [2, 3]], jnp.int32), ((0, 0), (0, 14)))
    attB = jnp.concatenate([att_s2, att_p2], axis=0)
    biasB = jnp.stack([b_s2, b_p2])
    flagB = jnp.ones((2, D), jnp.float32)
    outB = _edge_phase(2, tblB, src_s, dst_s, ebm, tixB, attB, biasB, flagB)

    return _finalize(outB[0], outB[1])


# speedup vs baseline: 2.3617x; 1.0170x over previous
import functools

import jax
import jax.numpy as jnp
from jax import lax
from jax.experimental import pallas as pl
from jax.experimental.pallas import tpu as pltpu
from jax.experimental.pallas import tpu_sc as plsc

N = 10000
E = 160000
IN_DIM = 256
HIDDEN = 128
HEADS = 4
S_DIM = 64
P_DIM = 64
ETA = 1e-6
NEG_SLOPE = 0.2

E2 = E + N
BE = 128
E2P = E2 + BE
R = 320
NW = 32
D = 128
NC = 2
BR = 1000


def _m1_body(x_ref, w_ref, b_ref, o_ref):
    acc = jnp.dot(x_ref[...], w_ref[...], preferred_element_type=jnp.float32)
    o_ref[0] = acc + b_ref[0, 0:1, :]


def _proj1(x, wcat, bcat, nt):
    k = x.shape[1]
    return pl.pallas_call(
        _m1_body,
        grid=(nt, N // BR),
        in_specs=[
            pl.BlockSpec((BR, k), lambda i, j: (j, 0)),
            pl.BlockSpec((k, D), lambda i, j: (0, i)),
            pl.BlockSpec((1, 8, D), lambda i, j: (i, 0, 0)),
        ],
        out_specs=pl.BlockSpec((1, BR, D), lambda i, j: (i, j, 0)),
        out_shape=jax.ShapeDtypeStruct((nt, N, D), jnp.float32),
    )(x, wcat, bcat)


def _m2_body(s_ref, w_ref, b_ref, o_ref):
    acc = jnp.dot(
        s_ref[0], w_ref[pl.ds(0, D), :], preferred_element_type=jnp.float32
    )
    for h in range(1, HEADS):
        acc += jnp.dot(
            s_ref[h], w_ref[pl.ds(h * D, D), :],
            preferred_element_type=jnp.float32,
        )
    o_ref[0] = acc + b_ref[0, 0:1, :]


def _proj2(s4, wcat, bcat):
    return pl.pallas_call(
        _m2_body,
        grid=(2, N // BR),
        in_specs=[
            pl.BlockSpec((HEADS, BR, D), lambda i, j: (0, j, 0)),
            pl.BlockSpec((HEADS * D, D), lambda i, j: (0, i)),
            pl.BlockSpec((1, 8, D), lambda i, j: (i, 0, 0)),
        ],
        out_specs=pl.BlockSpec((1, BR, D), lambda i, j: (i, j, 0)),
        out_shape=jax.ShapeDtypeStruct((2, N, D), jnp.float32),
    )(s4, wcat, bcat)


def _fin_body(s_ref, p_ref, o1_ref, o2_ref):
    o1_ref[:, :S_DIM] = s_ref[:, :S_DIM]
    o1_ref[:, S_DIM:] = p_ref[:, :P_DIM]
    o2_ref[:, :S_DIM] = jax.nn.softplus(s_ref[:, S_DIM:]) + ETA
    o2_ref[:, S_DIM:] = jax.nn.softplus(p_ref[:, P_DIM:]) + ETA


def _finalize(s, p):
    return pl.pallas_call(
        _fin_body,
        grid=(N // BR,),
        in_specs=[
            pl.BlockSpec((BR, D), lambda i: (i, 0)),
            pl.BlockSpec((BR, D), lambda i: (i, 0)),
        ],
        out_specs=[
            pl.BlockSpec((BR, D), lambda i: (i, 0)),
            pl.BlockSpec((BR, D), lambda i: (i, 0)),
        ],
        out_shape=[
            jax.ShapeDtypeStruct((N, D), jnp.float32),
            jax.ShapeDtypeStruct((N, D), jnp.float32),
        ],
    )(s, p)


def _edge_body(hp_n, tbl_ref, src_ref, dst_ref, eb_ref, tix_ref,
               att_ref, bias_ref, flag_ref, out_ref,
               ebv, tixv, attv, biasv, flagv,
               xrv, accv, denv, srcb, dstb, gidx, msgv, sem):
    wid = lax.axis_index("s") * NC + lax.axis_index("c")
    base_own = wid * R
    base_x = jnp.minimum(base_own, N - R)
    pltpu.sync_copy(eb_ref.at[wid], ebv)
    ebvec = ebv[...]
    e0 = ebvec[0] & ~jnp.int32(BE - 1)
    e1 = ebvec[1]
    nb = (e1 - e0 + (BE - 1)) // BE

    def hp_loop(hp, _):
        pltpu.sync_copy(tix_ref.at[hp], tixv)
        tvec = tixv[...]
        xli = tvec[0]
        xri = tvec[1]
        pltpu.sync_copy(att_ref.at[hp], attv)
        pltpu.sync_copy(bias_ref.at[hp], biasv)
        pltpu.sync_copy(flag_ref.at[hp], flagv)
        pltpu.sync_copy(tbl_ref.at[pl.ds(pl.multiple_of(xri * N + base_x, 8), R), :], xrv)

        zero16 = jnp.zeros((16,), jnp.float32)

        def zero_loop(r, _):
            for c in range(8):
                accv[r, pl.ds(c * 16, 16)] = zero16
            return 0

        lax.fori_loop(0, R, zero_loop, 0)
        for i in range(20):
            denv[pl.ds(i * 16, 16)] = zero16

        tbase = xli * N

        def blk_loop(b, _):
            off = pl.multiple_of(e0 + b * BE, BE)
            pltpu.sync_copy(src_ref.at[pl.ds(off, BE)], srcb)
            pltpu.sync_copy(dst_ref.at[pl.ds(off, BE)], dstb)
            for c in range(8):
                gidx[pl.ds(c * 16, 16)] = srcb[pl.ds(c * 16, 16)] + tbase
            pltpu.async_copy(tbl_ref.at[gidx], msgv, sem).wait()
            for c in range(8):
                erow = lax.iota(jnp.int32, 16) + (c * 16)
                d16 = dstb[pl.ds(c * 16, 16)]
                mask = (d16 >= base_own) & (d16 < base_own + R)
                dl = jnp.clip(d16 - base_x, 0, R - 1)

                def alpha_loop(k8, alpha):
                    kb = k8 * 8
                    for j in range(8):
                        colk = jnp.full((16,), kb + j, jnp.int32)
                        mv = plsc.load_gather(msgv, [erow, colk])
                        xv = plsc.load_gather(xrv, [dl, colk])
                        v = mv + xv
                        v = jnp.where(v > 0, v, NEG_SLOPE * v)
                        av = plsc.load_gather(attv, [colk])
                        alpha = alpha + v * av
                    return alpha

                alpha = lax.fori_loop(0, D // 8, alpha_loop, zero16)
                w = jnp.where(mask, jnp.exp(alpha), 0.0)
                plsc.addupdate_scatter(denv, [dl], w, mask=mask)

                def acc_loop(k8, _):
                    kb = k8 * 8
                    for j in range(8):
                        colk = jnp.full((16,), kb + j, jnp.int32)
                        mv = plsc.load_gather(msgv, [erow, colk])
                        plsc.addupdate_scatter(
                            accv, [dl, colk], w * mv, mask=mask
                        )
                    return 0

                lax.fori_loop(0, D // 8, acc_loop, 0)
            return 0

        lax.fori_loop(0, nb, blk_loop, 0)

        def fin_loop(r, _):
            r16 = jnp.full((16,), r, jnp.int32)
            rin = 1.0 / plsc.load_gather(denv, [r16])
            for c in range(8):
                v = accv[r, pl.ds(c * 16, 16)]
                bv = biasv[pl.ds(c * 16, 16)]
                fv = flagv[pl.ds(c * 16, 16)]
                v = v * rin + bv
                accv[r, pl.ds(c * 16, 16)] = jnp.maximum(v, fv * v)
            return 0

        lax.fori_loop(0, R, fin_loop, 0)

        @pl.when(wid < NW - 1)
        def _():
            pltpu.sync_copy(accv, out_ref.at[hp, pl.ds(pl.multiple_of(base_own, 8), R), :])

        @pl.when(wid == NW - 1)
        def _():
            nlast = N - (NW - 1) * R
            pltpu.sync_copy(
                accv.at[pl.ds(R - nlast, nlast), :],
                out_ref.at[hp, pl.ds((NW - 1) * R, nlast), :],
            )

        return 0

    lax.fori_loop(0, hp_n, hp_loop, 0)


def _edge_phase(hp_n, tbl, src_s, dst_s, eb, tix, att, bias, flag):
    mesh = plsc.VectorSubcoreMesh(core_axis_name="c", subcore_axis_name="s")
    fn = pl.kernel(
        functools.partial(_edge_body, hp_n),
        mesh=mesh,
        compiler_params=pltpu.CompilerParams(needs_layout_passes=False),
        out_type=jax.ShapeDtypeStruct((hp_n, N, D), jnp.float32),
        scratch_types=[
            pltpu.VMEM((16,), jnp.int32),
            pltpu.VMEM((16,), jnp.int32),
            pltpu.VMEM((D,), jnp.float32),
            pltpu.VMEM((D,), jnp.float32),
            pltpu.VMEM((D,), jnp.float32),
            pltpu.VMEM((R, D), jnp.float32),
            pltpu.VMEM((R, D), jnp.float32),
            pltpu.VMEM((320,), jnp.float32),
            pltpu.VMEM((BE,), jnp.int32),
            pltpu.VMEM((BE,), jnp.int32),
            pltpu.VMEM((BE,), jnp.int32),
            pltpu.VMEM((BE, D), jnp.float32),
            pltpu.SemaphoreType.DMA,
        ],
    )
    return fn(tbl, src_s, dst_s, eb, tix, att, bias, flag)


def kernel(x, edge_index, Wl_s1, bl_s1, Wr_s1, br_s1, att_s1, b_s1, Wl_s2, bl_s2, Wr_s2, br_s2, att_s2, b_s2, Wl_p1, bl_p1, Wr_p1, br_p1, att_p1, b_p1, Wl_p2, bl_p2, Wr_p2, br_p2, att_p2, b_p2):
    loop = jnp.arange(N, dtype=edge_index.dtype)
    src = jnp.concatenate([edge_index[0], loop])
    dst = jnp.concatenate([edge_index[1], loop])
    perm = jnp.argsort(dst)
    dst_s = dst[perm]
    src_s = src[perm]
    src_s = jnp.concatenate([src_s, jnp.zeros((E2P - E2,), jnp.int32)])
    dst_s = jnp.concatenate(
        [dst_s, jnp.full((E2P - E2,), jnp.int32(1 << 20))]
    )
    bounds = jnp.arange(0, NW + 1, dtype=jnp.int32) * R
    eb = jnp.searchsorted(dst_s[:E2], bounds).astype(jnp.int32)
    ebm = jnp.stack([eb[:NW], eb[1:]], axis=1)
    ebm = jnp.pad(ebm, ((0, 0), (0, 14)))

    w1 = jnp.concatenate([Wl_s1, Wr_s1, Wl_p1, Wr_p1], axis=1)
    b1 = jnp.concatenate([bl_s1, br_s1, bl_p1, br_p1]).reshape(16, 1, D)
    b1 = jnp.broadcast_to(b1, (16, 8, D))
    h1 = _proj1(x, w1, b1, 16)

    tixA = jnp.pad(
        jnp.array(
            [[0, 4], [1, 5], [2, 6], [3, 7],
             [8, 12], [9, 13], [10, 14], [11, 15]], jnp.int32
        ),
        ((0, 0), (0, 14)),
    )
    attA = jnp.concatenate([att_s1, att_p1], axis=0)
    biasA = jnp.concatenate([b_s1.reshape(4, D), b_p1.reshape(4, D)], axis=0)
    flagA = jnp.concatenate(
        [jnp.zeros((4, D), jnp.float32), jnp.ones((4, D), jnp.float32)]
    )
    outA = _edge_phase(
        8, h1.reshape(16 * N, D), src_s, dst_s, ebm, tixA, attA, biasA, flagA
    )

    w2s = jnp.concatenate([Wl_s2, Wr_s2], axis=1)
    b2s = jnp.concatenate([bl_s2, br_s2]).reshape(2, 1, D)
    b2s = jnp.broadcast_to(b2s, (2, 8, D))
    w2p = jnp.concatenate([Wl_p2, Wr_p2], axis=1)
    b2p = jnp.concatenate([bl_p2, br_p2]).reshape(2, 1, D)
    b2p = jnp.broadcast_to(b2p, (2, 8, D))
    s2 = _proj2(outA[0:4], w2s, b2s)
    p2 = _proj2(outA[4:8], w2p, b2p)
    tblB = jnp.concatenate([s2, p2], axis=0).reshape(4 * N, D)

    tixB = jnp.pad(jnp.array([[0, 1], [2, 3]], jnp.int32), ((0, 0), (0, 14)))
    attB = jnp.concatenate([att_s2, att_p2], axis=0)
    biasB = jnp.stack([b_s2, b_p2])
    flagB = jnp.ones((2, D), jnp.float32)
    outB = _edge_phase(2, tblB, src_s, dst_s, ebm, tixB, attB, biasB, flagB)

    return _finalize(outB[0], outB[1])


# speedup vs baseline: 2.4524x; 1.0384x over previous
import functools

import jax
import jax.numpy as jnp
from jax import lax
from jax.experimental import pallas as pl
from jax.experimental.pallas import tpu as pltpu
from jax.experimental.pallas import tpu_sc as plsc

N = 10000
E = 160000
IN_DIM = 256
HIDDEN = 128
HEADS = 4
S_DIM = 64
P_DIM = 64
ETA = 1e-6
NEG_SLOPE = 0.2

E2 = E + N
BE = 128
E2P = E2 + 5 * BE
R = 320
NW = 32
D = 128
NC = 2
BR = 1000


def _m1_body(x_ref, w_ref, b_ref, o_ref):
    acc = jnp.dot(x_ref[...], w_ref[...], preferred_element_type=jnp.float32)
    o_ref[0] = acc + b_ref[0, 0:1, :]


def _proj1(x, wcat, bcat, nt):
    k = x.shape[1]
    return pl.pallas_call(
        _m1_body,
        grid=(nt, N // BR),
        in_specs=[
            pl.BlockSpec((BR, k), lambda i, j: (j, 0)),
            pl.BlockSpec((k, D), lambda i, j: (0, i)),
            pl.BlockSpec((1, 8, D), lambda i, j: (i, 0, 0)),
        ],
        out_specs=pl.BlockSpec((1, BR, D), lambda i, j: (i, j, 0)),
        out_shape=jax.ShapeDtypeStruct((nt, N, D), jnp.float32),
    )(x, wcat, bcat)


def _m2_body(s_ref, w_ref, b_ref, o_ref):
    acc = jnp.dot(
        s_ref[0], w_ref[pl.ds(0, D), :], preferred_element_type=jnp.float32
    )
    for h in range(1, HEADS):
        acc += jnp.dot(
            s_ref[h], w_ref[pl.ds(h * D, D), :],
            preferred_element_type=jnp.float32,
        )
    o_ref[0] = acc + b_ref[0, 0:1, :]


def _proj2(s4, wcat, bcat):
    return pl.pallas_call(
        _m2_body,
        grid=(2, N // BR),
        in_specs=[
            pl.BlockSpec((HEADS, BR, D), lambda i, j: (0, j, 0)),
            pl.BlockSpec((HEADS * D, D), lambda i, j: (0, i)),
            pl.BlockSpec((1, 8, D), lambda i, j: (i, 0, 0)),
        ],
        out_specs=pl.BlockSpec((1, BR, D), lambda i, j: (i, j, 0)),
        out_shape=jax.ShapeDtypeStruct((2, N, D), jnp.float32),
    )(s4, wcat, bcat)


def _fin_body(s_ref, p_ref, o1_ref, o2_ref):
    o1_ref[:, :S_DIM] = s_ref[:, :S_DIM]
    o1_ref[:, S_DIM:] = p_ref[:, :P_DIM]
    o2_ref[:, :S_DIM] = jax.nn.softplus(s_ref[:, S_DIM:]) + ETA
    o2_ref[:, S_DIM:] = jax.nn.softplus(p_ref[:, P_DIM:]) + ETA


def _finalize(s, p):
    return pl.pallas_call(
        _fin_body,
        grid=(N // BR,),
        in_specs=[
            pl.BlockSpec((BR, D), lambda i: (i, 0)),
            pl.BlockSpec((BR, D), lambda i: (i, 0)),
        ],
        out_specs=[
            pl.BlockSpec((BR, D), lambda i: (i, 0)),
            pl.BlockSpec((BR, D), lambda i: (i, 0)),
        ],
        out_shape=[
            jax.ShapeDtypeStruct((N, D), jnp.float32),
            jax.ShapeDtypeStruct((N, D), jnp.float32),
        ],
    )(s, p)


def _edge_body(hp_n, tbl_ref, src_ref, dst_ref, eb_ref, tix_ref,
               att_ref, bias_ref, flag_ref, out_ref,
               ebv, tixv, attv, biasv, flagv,
               xrv, accv, denv,
               srcb0, dstb0, gidx0, msg0,
               srcb1, dstb1, gidx1, msg1,
               sem0, sem1):
    wid = lax.axis_index("s") * NC + lax.axis_index("c")
    base_own = wid * R
    base_x = jnp.minimum(base_own, N - R)
    pltpu.sync_copy(eb_ref.at[wid], ebv)
    ebvec = ebv[...]
    e0 = ebvec[0] & ~jnp.int32(BE - 1)
    e1 = ebvec[1]
    nb = (e1 - e0 + (BE - 1)) // BE

    def hp_loop(hp, _):
        pltpu.sync_copy(tix_ref.at[hp], tixv)
        tvec = tixv[...]
        xli = tvec[0]
        xri = tvec[1]
        pltpu.sync_copy(att_ref.at[hp], attv)
        pltpu.sync_copy(bias_ref.at[hp], biasv)
        pltpu.sync_copy(flag_ref.at[hp], flagv)
        pltpu.sync_copy(tbl_ref.at[pl.ds(pl.multiple_of(xri * N + base_x, 8), R), :], xrv)

        zero16 = jnp.zeros((16,), jnp.float32)

        def zero_loop(r, _):
            for c in range(8):
                accv[r, pl.ds(c * 16, 16)] = zero16
            return 0

        lax.fori_loop(0, R, zero_loop, 0)
        for i in range(20):
            denv[pl.ds(i * 16, 16)] = zero16

        tbase = xli * N

        def load_idx(bi, srcb, dstb, gidx):
            off = pl.multiple_of(e0 + bi * BE, BE)
            pltpu.sync_copy(src_ref.at[pl.ds(off, BE)], srcb)
            pltpu.sync_copy(dst_ref.at[pl.ds(off, BE)], dstb)
            for c in range(8):
                gidx[pl.ds(c * 16, 16)] = srcb[pl.ds(c * 16, 16)] + tbase

        def compute(dstb, msgv):
            for c in range(8):
                erow = lax.iota(jnp.int32, 16) + (c * 16)
                d16 = dstb[pl.ds(c * 16, 16)]
                mask = (d16 >= base_own) & (d16 < base_own + R)
                dl = jnp.clip(d16 - base_x, 0, R - 1)

                def alpha_loop(k8, accs):
                    kb = k8 * 8
                    out = []
                    for j in range(8):
                        colk = jnp.full((16,), kb + j, jnp.int32)
                        mv = plsc.load_gather(msgv, [erow, colk])
                        xv = plsc.load_gather(xrv, [dl, colk])
                        v = mv + xv
                        v = jnp.where(v > 0, v, NEG_SLOPE * v)
                        av = plsc.load_gather(attv, [colk])
                        out.append(accs[j] + v * av)
                    return tuple(out)

                accs = lax.fori_loop(0, D // 8, alpha_loop, (zero16,) * 8)
                alpha = ((accs[0] + accs[1]) + (accs[2] + accs[3])) + (
                    (accs[4] + accs[5]) + (accs[6] + accs[7])
                )
                w = jnp.where(mask, jnp.exp(alpha), 0.0)
                plsc.addupdate_scatter(denv, [dl], w, mask=mask)

                def acc_loop(k8, _):
                    kb = k8 * 8
                    for j in range(8):
                        colk = jnp.full((16,), kb + j, jnp.int32)
                        mv = plsc.load_gather(msgv, [erow, colk])
                        plsc.addupdate_scatter(
                            accv, [dl, colk], w * mv, mask=mask
                        )
                    return 0

                lax.fori_loop(0, D // 8, acc_loop, 0)

        load_idx(0, srcb0, dstb0, gidx0)
        pltpu.async_copy(tbl_ref.at[gidx0], msg0, sem0)
        load_idx(1, srcb1, dstb1, gidx1)
        pltpu.async_copy(tbl_ref.at[gidx1], msg1, sem1)

        def pair_loop(p, _):
            pltpu.make_async_copy(tbl_ref.at[gidx0], msg0, sem0).wait()
            compute(dstb0, msg0)
            load_idx(2 * p + 2, srcb0, dstb0, gidx0)
            pltpu.async_copy(tbl_ref.at[gidx0], msg0, sem0)
            pltpu.make_async_copy(tbl_ref.at[gidx1], msg1, sem1).wait()
            compute(dstb1, msg1)
            load_idx(2 * p + 3, srcb1, dstb1, gidx1)
            pltpu.async_copy(tbl_ref.at[gidx1], msg1, sem1)
            return 0

        lax.fori_loop(0, (nb + 1) // 2, pair_loop, 0)
        pltpu.make_async_copy(tbl_ref.at[gidx0], msg0, sem0).wait()
        pltpu.make_async_copy(tbl_ref.at[gidx1], msg1, sem1).wait()

        def fin_loop(r, _):
            r16 = jnp.full((16,), r, jnp.int32)
            rin = 1.0 / plsc.load_gather(denv, [r16])
            for c in range(8):
                v = accv[r, pl.ds(c * 16, 16)]
                bv = biasv[pl.ds(c * 16, 16)]
                fv = flagv[pl.ds(c * 16, 16)]
                v = v * rin + bv
                accv[r, pl.ds(c * 16, 16)] = jnp.maximum(v, fv * v)
            return 0

        lax.fori_loop(0, R, fin_loop, 0)

        @pl.when(wid < NW - 1)
        def _():
            pltpu.sync_copy(accv, out_ref.at[hp, pl.ds(pl.multiple_of(base_own, 8), R), :])

        @pl.when(wid == NW - 1)
        def _():
            nlast = N - (NW - 1) * R
            pltpu.sync_copy(
                accv.at[pl.ds(R - nlast, nlast), :],
                out_ref.at[hp, pl.ds((NW - 1) * R, nlast), :],
            )

        return 0

    lax.fori_loop(0, hp_n, hp_loop, 0)


def _edge_phase(hp_n, tbl, src_s, dst_s, eb, tix, att, bias, flag):
    mesh = plsc.VectorSubcoreMesh(core_axis_name="c", subcore_axis_name="s")
    fn = pl.kernel(
        functools.partial(_edge_body, hp_n),
        mesh=mesh,
        compiler_params=pltpu.CompilerParams(needs_layout_passes=False),
        out_type=jax.ShapeDtypeStruct((hp_n, N, D), jnp.float32),
        scratch_types=[
            pltpu.VMEM((16,), jnp.int32),
            pltpu.VMEM((16,), jnp.int32),
            pltpu.VMEM((D,), jnp.float32),
            pltpu.VMEM((D,), jnp.float32),
            pltpu.VMEM((D,), jnp.float32),
            pltpu.VMEM((R, D), jnp.float32),
            pltpu.VMEM((R, D), jnp.float32),
            pltpu.VMEM((320,), jnp.float32),
            pltpu.VMEM((BE,), jnp.int32),
            pltpu.VMEM((BE,), jnp.int32),
            pltpu.VMEM((BE,), jnp.int32),
            pltpu.VMEM((BE, D), jnp.float32),
            pltpu.VMEM((BE,), jnp.int32),
            pltpu.VMEM((BE,), jnp.int32),
            pltpu.VMEM((BE,), jnp.int32),
            pltpu.VMEM((BE, D), jnp.float32),
            pltpu.SemaphoreType.DMA,
            pltpu.SemaphoreType.DMA,
        ],
    )
    return fn(tbl, src_s, dst_s, eb, tix, att, bias, flag)


def kernel(x, edge_index, Wl_s1, bl_s1, Wr_s1, br_s1, att_s1, b_s1, Wl_s2, bl_s2, Wr_s2, br_s2, att_s2, b_s2, Wl_p1, bl_p1, Wr_p1, br_p1, att_p1, b_p1, Wl_p2, bl_p2, Wr_p2, br_p2, att_p2, b_p2):
    loop = jnp.arange(N, dtype=edge_index.dtype)
    src = jnp.concatenate([edge_index[0], loop])
    dst = jnp.concatenate([edge_index[1], loop])
    perm = jnp.argsort(dst)
    dst_s = dst[perm]
    src_s = src[perm]
    src_s = jnp.concatenate([src_s, jnp.zeros((E2P - E2,), jnp.int32)])
    dst_s = jnp.concatenate(
        [dst_s, jnp.full((E2P - E2,), jnp.int32(1 << 20))]
    )
    bounds = jnp.arange(0, NW + 1, dtype=jnp.int32) * R
    eb = jnp.searchsorted(dst_s[:E2], bounds).astype(jnp.int32)
    ebm = jnp.stack([eb[:NW], eb[1:]], axis=1)
    ebm = jnp.pad(ebm, ((0, 0), (0, 14)))

    w1 = jnp.concatenate([Wl_s1, Wr_s1, Wl_p1, Wr_p1], axis=1)
    b1 = jnp.concatenate([bl_s1, br_s1, bl_p1, br_p1]).reshape(16, 1, D)
    b1 = jnp.broadcast_to(b1, (16, 8, D))
    h1 = _proj1(x, w1, b1, 16)

    tixA = jnp.pad(
        jnp.array(
            [[0, 4], [1, 5], [2, 6], [3, 7],
             [8, 12], [9, 13], [10, 14], [11, 15]], jnp.int32
        ),
        ((0, 0), (0, 14)),
    )
    attA = jnp.concatenate([att_s1, att_p1], axis=0)
    biasA = jnp.concatenate([b_s1.reshape(4, D), b_p1.reshape(4, D)], axis=0)
    flagA = jnp.concatenate(
        [jnp.zeros((4, D), jnp.float32), jnp.ones((4, D), jnp.float32)]
    )
    outA = _edge_phase(
        8, h1.reshape(16 * N, D), src_s, dst_s, ebm, tixA, attA, biasA, flagA
    )

    w2s = jnp.concatenate([Wl_s2, Wr_s2], axis=1)
    b2s = jnp.concatenate([bl_s2, br_s2]).reshape(2, 1, D)
    b2s = jnp.broadcast_to(b2s, (2, 8, D))
    w2p = jnp.concatenate([Wl_p2, Wr_p2], axis=1)
    b2p = jnp.concatenate([bl_p2, br_p2]).reshape(2, 1, D)
    b2p = jnp.broadcast_to(b2p, (2, 8, D))
    s2 = _proj2(outA[0:4], w2s, b2s)
    p2 = _proj2(outA[4:8], w2p, b2p)
    tblB = jnp.concatenate([s2, p2], axis=0).reshape(4 * N, D)

    tixB = jnp.pad(jnp.array([[0, 1], [2, 3]], jnp.int32), ((0, 0), (0, 14)))
    attB = jnp.concatenate([att_s2, att_p2], axis=0)
    biasB = jnp.stack([b_s2, b_p2])
    flagB = jnp.ones((2, D), jnp.float32)
    outB = _edge_phase(2, tblB, src_s, dst_s, ebm, tixB, attB, biasB, flagB)

    return _finalize(outB[0], outB[1])


# speedup vs baseline: 4.6006x; 1.8760x over previous
import functools

import jax
import jax.numpy as jnp
from jax import lax
from jax.experimental import pallas as pl
from jax.experimental.pallas import tpu as pltpu
from jax.experimental.pallas import tpu_sc as plsc

N = 10000
E = 160000
IN_DIM = 256
HIDDEN = 128
HEADS = 4
S_DIM = 64
P_DIM = 64
ETA = 1e-6
NEG_SLOPE = 0.2

E2 = E + N
BE = 128
E2P = E2 + 5 * BE
R = 320
NW = 32
D = 128
NC = 2
BR = 1000


def _m1_body(x_ref, w_ref, b_ref, o_ref):
    acc = jnp.dot(x_ref[...], w_ref[...], preferred_element_type=jnp.float32)
    o_ref[0] = acc + b_ref[0, 0:1, :]


def _proj1(x, wcat, bcat, nt):
    k = x.shape[1]
    return pl.pallas_call(
        _m1_body,
        grid=(nt, N // BR),
        in_specs=[
            pl.BlockSpec((BR, k), lambda i, j: (j, 0)),
            pl.BlockSpec((k, D), lambda i, j: (0, i)),
            pl.BlockSpec((1, 8, D), lambda i, j: (i, 0, 0)),
        ],
        out_specs=pl.BlockSpec((1, BR, D), lambda i, j: (i, j, 0)),
        out_shape=jax.ShapeDtypeStruct((nt, N, D), jnp.float32),
    )(x, wcat, bcat)


def _m2_body(s_ref, w_ref, b_ref, o_ref):
    acc = jnp.dot(
        s_ref[0], w_ref[pl.ds(0, D), :], preferred_element_type=jnp.float32
    )
    for h in range(1, HEADS):
        acc += jnp.dot(
            s_ref[h], w_ref[pl.ds(h * D, D), :],
            preferred_element_type=jnp.float32,
        )
    o_ref[0] = acc + b_ref[0, 0:1, :]


def _proj2(s4, wcat, bcat):
    return pl.pallas_call(
        _m2_body,
        grid=(2, N // BR),
        in_specs=[
            pl.BlockSpec((HEADS, BR, D), lambda i, j: (0, j, 0)),
            pl.BlockSpec((HEADS * D, D), lambda i, j: (0, i)),
            pl.BlockSpec((1, 8, D), lambda i, j: (i, 0, 0)),
        ],
        out_specs=pl.BlockSpec((1, BR, D), lambda i, j: (i, j, 0)),
        out_shape=jax.ShapeDtypeStruct((2, N, D), jnp.float32),
    )(s4, wcat, bcat)


def _fin_body(s_ref, p_ref, o1_ref, o2_ref):
    o1_ref[:, :S_DIM] = s_ref[:, :S_DIM]
    o1_ref[:, S_DIM:] = p_ref[:, :P_DIM]
    o2_ref[:, :S_DIM] = jax.nn.softplus(s_ref[:, S_DIM:]) + ETA
    o2_ref[:, S_DIM:] = jax.nn.softplus(p_ref[:, P_DIM:]) + ETA


def _finalize(s, p):
    return pl.pallas_call(
        _fin_body,
        grid=(N // BR,),
        in_specs=[
            pl.BlockSpec((BR, D), lambda i: (i, 0)),
            pl.BlockSpec((BR, D), lambda i: (i, 0)),
        ],
        out_specs=[
            pl.BlockSpec((BR, D), lambda i: (i, 0)),
            pl.BlockSpec((BR, D), lambda i: (i, 0)),
        ],
        out_shape=[
            jax.ShapeDtypeStruct((N, D), jnp.float32),
            jax.ShapeDtypeStruct((N, D), jnp.float32),
        ],
    )(s, p)


def _edge_body(hp_n, tbl_ref, src_ref, dst_ref, eb_ref, tix_ref,
               att_ref, bias_ref, flag_ref, out_ref,
               ebv, tixv, attv, biasv, flagv,
               xrv, accv, denv,
               srcb0, dstb0, gidx0, msg0,
               srcb1, dstb1, gidx1, msg1,
               sem0, sem1):
    wid = lax.axis_index("s") * NC + lax.axis_index("c")
    base_own = wid * R
    base_x = jnp.minimum(base_own, N - R)
    pltpu.sync_copy(eb_ref.at[wid], ebv)
    ebvec = ebv[...]
    e0 = ebvec[0] & ~jnp.int32(BE - 1)
    e1 = ebvec[1]
    nb = (e1 - e0 + (BE - 1)) // BE

    def hp_loop(hp, _):
        pltpu.sync_copy(tix_ref.at[hp], tixv)
        tvec = tixv[...]
        xli = tvec[0]
        xri = tvec[1]
        pltpu.sync_copy(att_ref.at[hp], attv)
        pltpu.sync_copy(bias_ref.at[hp], biasv)
        pltpu.sync_copy(flag_ref.at[hp], flagv)
        pltpu.sync_copy(tbl_ref.at[pl.ds(pl.multiple_of(xri * N + base_x, 8), R), :], xrv)

        zero16 = jnp.zeros((16,), jnp.float32)

        def zero_loop(r, _):
            for c in range(8):
                accv[r, pl.ds(c * 16, 16)] = zero16
            return 0

        lax.fori_loop(0, R, zero_loop, 0)
        for i in range(20):
            denv[pl.ds(i * 16, 16)] = zero16

        tbase = xli * N

        def load_idx(bi, srcb, dstb, gidx):
            off = pl.multiple_of(e0 + bi * BE, BE)
            pltpu.sync_copy(src_ref.at[pl.ds(off, BE)], srcb)
            off16 = pl.multiple_of(off // 16, 8)
            pltpu.sync_copy(dst_ref.at[pl.ds(off16, 8), :], dstb)
            for c in range(8):
                gidx[pl.ds(c * 16, 16)] = srcb[pl.ds(c * 16, 16)] + tbase

        def compute(dstb, msgv):
            def c_loop(c, _):
                erow = lax.iota(jnp.int32, 16) + c * 16
                d16 = dstb[c, :]
                mask = (d16 >= base_own) & (d16 < base_own + R)
                dl = jnp.clip(d16 - base_x, 0, R - 1)

                def alpha_loop(k8, accs):
                    kb = k8 * 8
                    out = []
                    for j in range(8):
                        colk = jnp.full((16,), kb + j, jnp.int32)
                        mv = plsc.load_gather(msgv, [erow, colk])
                        xv = plsc.load_gather(xrv, [dl, colk])
                        v = mv + xv
                        v = jnp.where(v > 0, v, NEG_SLOPE * v)
                        av = plsc.load_gather(attv, [colk])
                        out.append(accs[j] + v * av)
                    return tuple(out)

                accs = lax.fori_loop(0, D // 8, alpha_loop, (zero16,) * 8)
                alpha = ((accs[0] + accs[1]) + (accs[2] + accs[3])) + (
                    (accs[4] + accs[5]) + (accs[6] + accs[7])
                )
                w = jnp.where(mask, jnp.exp(alpha), 0.0)
                plsc.addupdate_scatter(denv, [dl], w, mask=mask)

                for e in range(16):
                    dle = dl[e]
                    wev = jnp.full((16,), w[e], jnp.float32)
                    row = c * 16 + e
                    for c8 in range(8):
                        cs = pl.ds(c8 * 16, 16)
                        accv[dle, cs] = (
                            accv[dle, cs] + wev * msgv[row, cs]
                        )
                return 0

            lax.fori_loop(0, 8, c_loop, 0)

        load_idx(0, srcb0, dstb0, gidx0)
        pltpu.async_copy(tbl_ref.at[gidx0], msg0, sem0)
        load_idx(1, srcb1, dstb1, gidx1)
        pltpu.async_copy(tbl_ref.at[gidx1], msg1, sem1)

        def pair_loop(p, _):
            pltpu.make_async_copy(tbl_ref.at[gidx0], msg0, sem0).wait()
            compute(dstb0, msg0)
            load_idx(2 * p + 2, srcb0, dstb0, gidx0)
            pltpu.async_copy(tbl_ref.at[gidx0], msg0, sem0)
            pltpu.make_async_copy(tbl_ref.at[gidx1], msg1, sem1).wait()
            compute(dstb1, msg1)
            load_idx(2 * p + 3, srcb1, dstb1, gidx1)
            pltpu.async_copy(tbl_ref.at[gidx1], msg1, sem1)
            return 0

        lax.fori_loop(0, (nb + 1) // 2, pair_loop, 0)
        pltpu.make_async_copy(tbl_ref.at[gidx0], msg0, sem0).wait()
        pltpu.make_async_copy(tbl_ref.at[gidx1], msg1, sem1).wait()

        def fin_loop(r, _):
            r16 = jnp.full((16,), r, jnp.int32)
            rin = 1.0 / plsc.load_gather(denv, [r16])
            for c in range(8):
                v = accv[r, pl.ds(c * 16, 16)]
                bv = biasv[pl.ds(c * 16, 16)]
                fv = flagv[pl.ds(c * 16, 16)]
                v = v * rin + bv
                accv[r, pl.ds(c * 16, 16)] = jnp.maximum(v, fv * v)
            return 0

        lax.fori_loop(0, R, fin_loop, 0)

        @pl.when(wid < NW - 1)
        def _():
            pltpu.sync_copy(accv, out_ref.at[hp, pl.ds(pl.multiple_of(base_own, 8), R), :])

        @pl.when(wid == NW - 1)
        def _():
            nlast = N - (NW - 1) * R
            pltpu.sync_copy(
                accv.at[pl.ds(R - nlast, nlast), :],
                out_ref.at[hp, pl.ds((NW - 1) * R, nlast), :],
            )

        return 0

    lax.fori_loop(0, hp_n, hp_loop, 0)


def _edge_phase(hp_n, tbl, src_s, dst_s, eb, tix, att, bias, flag):
    mesh = plsc.VectorSubcoreMesh(core_axis_name="c", subcore_axis_name="s")
    fn = pl.kernel(
        functools.partial(_edge_body, hp_n),
        mesh=mesh,
        compiler_params=pltpu.CompilerParams(needs_layout_passes=False),
        out_type=jax.ShapeDtypeStruct((hp_n, N, D), jnp.float32),
        scratch_types=[
            pltpu.VMEM((16,), jnp.int32),
            pltpu.VMEM((16,), jnp.int32),
            pltpu.VMEM((D,), jnp.float32),
            pltpu.VMEM((D,), jnp.float32),
            pltpu.VMEM((D,), jnp.float32),
            pltpu.VMEM((R, D), jnp.float32),
            pltpu.VMEM((R, D), jnp.float32),
            pltpu.VMEM((320,), jnp.float32),
            pltpu.VMEM((BE,), jnp.int32),
            pltpu.VMEM((8, 16), jnp.int32),
            pltpu.VMEM((BE,), jnp.int32),
            pltpu.VMEM((BE, D), jnp.float32),
            pltpu.VMEM((BE,), jnp.int32),
            pltpu.VMEM((8, 16), jnp.int32),
            pltpu.VMEM((BE,), jnp.int32),
            pltpu.VMEM((BE, D), jnp.float32),
            pltpu.SemaphoreType.DMA,
            pltpu.SemaphoreType.DMA,
        ],
    )
    return fn(tbl, src_s, dst_s, eb, tix, att, bias, flag)


def kernel(x, edge_index, Wl_s1, bl_s1, Wr_s1, br_s1, att_s1, b_s1, Wl_s2, bl_s2, Wr_s2, br_s2, att_s2, b_s2, Wl_p1, bl_p1, Wr_p1, br_p1, att_p1, b_p1, Wl_p2, bl_p2, Wr_p2, br_p2, att_p2, b_p2):
    loop = jnp.arange(N, dtype=edge_index.dtype)
    src = jnp.concatenate([edge_index[0], loop])
    dst = jnp.concatenate([edge_index[1], loop])
    perm = jnp.argsort(dst)
    dst_s = dst[perm]
    src_s = src[perm]
    src_s = jnp.concatenate([src_s, jnp.zeros((E2P - E2,), jnp.int32)])
    dst_s = jnp.concatenate(
        [dst_s, jnp.full((E2P - E2,), jnp.int32(1 << 20))]
    )
    dst2d = dst_s.reshape(E2P // 16, 16)
    bounds = jnp.arange(0, NW + 1, dtype=jnp.int32) * R
    eb = jnp.searchsorted(dst_s[:E2], bounds).astype(jnp.int32)
    ebm = jnp.stack([eb[:NW], eb[1:]], axis=1)
    ebm = jnp.pad(ebm, ((0, 0), (0, 14)))

    w1 = jnp.concatenate([Wl_s1, Wr_s1, Wl_p1, Wr_p1], axis=1)
    b1 = jnp.concatenate([bl_s1, br_s1, bl_p1, br_p1]).reshape(16, 1, D)
    b1 = jnp.broadcast_to(b1, (16, 8, D))
    h1 = _proj1(x, w1, b1, 16)

    tixA = jnp.pad(
        jnp.array(
            [[0, 4], [1, 5], [2, 6], [3, 7],
             [8, 12], [9, 13], [10, 14], [11, 15]], jnp.int32
        ),
        ((0, 0), (0, 14)),
    )
    attA = jnp.concatenate([att_s1, att_p1], axis=0)
    biasA = jnp.concatenate([b_s1.reshape(4, D), b_p1.reshape(4, D)], axis=0)
    flagA = jnp.concatenate(
        [jnp.zeros((4, D), jnp.float32), jnp.ones((4, D), jnp.float32)]
    )
    outA = _edge_phase(
        8, h1.reshape(16 * N, D), src_s, dst2d, ebm, tixA, attA, biasA, flagA
    )

    w2s = jnp.concatenate([Wl_s2, Wr_s2], axis=1)
    b2s = jnp.concatenate([bl_s2, br_s2]).reshape(2, 1, D)
    b2s = jnp.broadcast_to(b2s, (2, 8, D))
    w2p = jnp.concatenate([Wl_p2, Wr_p2], axis=1)
    b2p = jnp.concatenate([bl_p2, br_p2]).reshape(2, 1, D)
    b2p = jnp.broadcast_to(b2p, (2, 8, D))
    s2 = _proj2(outA[0:4], w2s, b2s)
    p2 = _proj2(outA[4:8], w2p, b2p)
    tblB = jnp.concatenate([s2, p2], axis=0).reshape(4 * N, D)

    tixB = jnp.pad(jnp.array([[0, 1], [2, 3]], jnp.int32), ((0, 0), (0, 14)))
    attB = jnp.concatenate([att_s2, att_p2], axis=0)
    biasB = jnp.stack([b_s2, b_p2])
    flagB = jnp.ones((2, D), jnp.float32)
    outB = _edge_phase(2, tblB, src_s, dst2d, ebm, tixB, attB, biasB, flagB)

    return _finalize(outB[0], outB[1])
